# Initial kernel scaffold; baseline (speedup 1.0000x reference)
import jax
import jax.numpy as jnp
from jax.experimental import pallas as pl


def kernel(x, edge_index, params):
    raise NotImplementedError("write your pallas kernel here")


# speedup vs baseline: 6.1221x; 6.1221x over previous
import functools

import jax
import jax.numpy as jnp
from jax import lax
from jax.experimental import pallas as pl
from jax.experimental.pallas import tpu as pltpu
from jax.experimental.pallas import tpu_sc as plsc

N = 10000
D = 128
E = 320000

NPAD = 10240
NW = 32
K = 128
CHUNKS = 80
EPT = K * CHUNKS
EPAD = NW * EPT
RB = 512
GRID = NPAD // RB

_PREC = lax.Precision.HIGHEST


def _ln_rows(x, g, b):
    m = jnp.mean(x, axis=-1, keepdims=True)
    v = jnp.mean((x - m) ** 2, axis=-1, keepdims=True)
    return (x - m) / jnp.sqrt(v + 1e-5) * g + b


def _mm(a, bt):
    return jnp.dot(a, bt, preferred_element_type=jnp.float32, precision=_PREC)


def _in_proj_body(x_ref, w1t, b1, w2t, b2, g, b, o_ref):
    h = _ln_rows(x_ref[...], g[...], b[...])
    h = jnp.maximum(_mm(h, w1t[...]) + b1[...], 0.0)
    o_ref[...] = _mm(h, w2t[...]) + b2[...]


def _layer_body(h_ref, ab_ref, w1t, b1, w2t, b2, g, b, o_ref):
    h = h_ref[...]
    z = ab_ref[0] + ab_ref[1] - h
    z = jnp.maximum(_mm(z, w1t[...]) + b1[...], 0.0)
    z = _mm(z, w2t[...]) + b2[...]
    z = _ln_rows(z, g[...], b[...])
    o_ref[...] = h + jnp.maximum(z, 0.0)


def _out_proj_body(h_ref, x_ref, w1t, b1, w2t, b2, g, b, o_ref):
    d = jnp.maximum(_mm(h_ref[...], w1t[...]) + b1[...], 0.0)
    d = _mm(d, w2t[...]) + b2[...]
    o_ref[...] = _ln_rows(x_ref[...] + d, g[...], b[...])


def _row_spec():
    return pl.BlockSpec((RB, None), lambda i: (i, 0))


def _full(shape):
    return pl.BlockSpec(shape, lambda i: tuple(0 for _ in shape))


def _tc_call(body, n_rows_inputs, weight_shapes, extra_specs=()):
    in_specs = [pl.BlockSpec((RB, D), lambda i: (i, 0)) for _ in range(n_rows_inputs)]
    in_specs += list(extra_specs)
    in_specs += [_full(s) for s in weight_shapes]
    return pl.pallas_call(
        body,
        grid=(GRID,),
        in_specs=in_specs,
        out_specs=pl.BlockSpec((RB, D), lambda i: (i, 0)),
        out_shape=jax.ShapeDtypeStruct((NPAD, D), jnp.float32),
    )


@functools.lru_cache(maxsize=1)
def _make_segsum():
    mesh = plsc.VectorSubcoreMesh(core_axis_name="c", subcore_axis_name="s",
                                  num_cores=2, num_subcores=16)

    @functools.partial(
        pl.kernel,
        mesh=mesh,
        out_type=jax.ShapeDtypeStruct((2, NPAD, D), jnp.float32),
        scratch_types=[
            pltpu.VMEM((CHUNKS, K), jnp.int32),
            pltpu.VMEM((CHUNKS, K), jnp.int32),
            pltpu.VMEM((K, D), jnp.float32),
            pltpu.VMEM_SHARED((NPAD, D), jnp.float32),
            pltpu.SemaphoreType.DMA,
        ],
    )
    def segsum(h_hbm, src_hbm, dst_hbm, out_hbm, src_v, dst_v, rows_v, acc, sem):
        cid = lax.axis_index("c")
        sid = lax.axis_index("s")
        wid = cid * 16 + sid
        rpt = NPAD // 16
        rbase = sid * rpt
        pltpu.sync_copy(h_hbm.at[pl.ds(rbase, rpt)], acc.at[pl.ds(rbase, rpt)])
        pltpu.sync_copy(src_hbm.at[wid], src_v)
        pltpu.sync_copy(dst_hbm.at[wid], dst_v)
        plsc.subcore_barrier()

        def body(j, carry):
            pltpu.async_copy(h_hbm.at[src_v.at[j]], rows_v, sem).wait()
            pltpu.sync_copy(rows_v, acc.at[dst_v.at[j]], add=True)
            return carry

        lax.fori_loop(0, CHUNKS, body, 0)
        plsc.subcore_barrier()
        pltpu.sync_copy(acc.at[pl.ds(rbase, rpt)],
                        out_hbm.at[cid, pl.ds(rbase, rpt)])

    return segsum


def kernel(x, edge_index, params):
    xp = jnp.pad(x, ((0, NPAD - N), (0, 0)))

    def row(v):
        return v.reshape(1, -1)

    p = params
    h = _tc_call(_in_proj_body, 1,
                 [(D, 512), (1, 512), (512, D), (1, D), (1, D), (1, D)])(
        xp, p['in_w1'].T, row(p['in_b1']), p['in_w2'].T, row(p['in_b2']),
        row(p['in_ln_g']), row(p['in_ln_b']))

    pad_len = EPAD - E
    pad_idx = N + (jnp.arange(pad_len, dtype=jnp.int32) % (NPAD - N))
    srcp = jnp.concatenate([edge_index[0], pad_idx]).reshape(NW, CHUNKS, K)
    dstp = jnp.concatenate([edge_index[1], pad_idx]).reshape(NW, CHUNKS, K)

    layer_call = _tc_call(
        _layer_body, 1,
        [(D, D), (1, D), (D, D), (1, D), (1, D), (1, D)],
        extra_specs=[pl.BlockSpec((2, RB, D), lambda i: (0, i, 0))])

    segsum = _make_segsum()
    for lp in p['layers']:
        ab = segsum(h, srcp, dstp)
        h = layer_call(h, ab, lp['w1'].T, row(lp['b1']), lp['w2'].T,
                       row(lp['b2']), row(lp['ln_g']), row(lp['ln_b']))

    out = _tc_call(_out_proj_body, 2,
                   [(D, 512), (1, 512), (512, D), (1, D), (1, D), (1, D)])(
        h, xp, p['d_w1'].T, row(p['d_b1']), p['d_w2'].T, row(p['d_b2']),
        row(p['out_ln_g']), row(p['out_ln_b']))
    return out[:N]


# speedup vs baseline: 8.2856x; 1.3534x over previous
import functools

import jax
import jax.numpy as jnp
from jax import lax
from jax.experimental import pallas as pl
from jax.experimental.pallas import tpu as pltpu
from jax.experimental.pallas import tpu_sc as plsc

N = 10000
D = 128
E = 320000

NPAD = 10240
NW = 32
K = 128
CHUNKS = 80
PHASES = 2
CPP = CHUNKS // PHASES
EPT = K * CHUNKS
EPAD = NW * EPT
RB = 512
GRID = NPAD // RB

_PREC = lax.Precision.HIGHEST


def _ln_rows(x, g, b):
    m = jnp.mean(x, axis=-1, keepdims=True)
    v = jnp.mean((x - m) ** 2, axis=-1, keepdims=True)
    return (x - m) / jnp.sqrt(v + 1e-5) * g + b


def _mm(a, bt):
    return jnp.dot(a, bt, preferred_element_type=jnp.float32, precision=_PREC)


def _in_proj_body(x_ref, w1t, b1, w2t, b2, g, b, o_ref):
    h = _ln_rows(x_ref[...], g[...], b[...])
    h = jnp.maximum(_mm(h, w1t[...]) + b1[...], 0.0)
    o_ref[...] = _mm(h, w2t[...]) + b2[...]


def _layer_body(h_ref, ab_ref, w1t, b1, w2t, b2, g, b, o_ref):
    h = h_ref[...]
    z = ab_ref[0] + ab_ref[1] - h
    z = jnp.maximum(_mm(z, w1t[...]) + b1[...], 0.0)
    z = _mm(z, w2t[...]) + b2[...]
    z = _ln_rows(z, g[...], b[...])
    o_ref[...] = h + jnp.maximum(z, 0.0)


def _out_proj_body(h_ref, x_ref, w1t, b1, w2t, b2, g, b, o_ref):
    d = jnp.maximum(_mm(h_ref[...], w1t[...]) + b1[...], 0.0)
    d = _mm(d, w2t[...]) + b2[...]
    o_ref[...] = _ln_rows(x_ref[...] + d, g[...], b[...])


def _row_spec():
    return pl.BlockSpec((RB, None), lambda i: (i, 0))


def _full(shape):
    return pl.BlockSpec(shape, lambda i: tuple(0 for _ in shape))


def _tc_call(body, n_rows_inputs, weight_shapes, extra_specs=()):
    in_specs = [pl.BlockSpec((RB, D), lambda i: (i, 0)) for _ in range(n_rows_inputs)]
    in_specs += list(extra_specs)
    in_specs += [_full(s) for s in weight_shapes]
    return pl.pallas_call(
        body,
        grid=(GRID,),
        in_specs=in_specs,
        out_specs=pl.BlockSpec((RB, D), lambda i: (i, 0)),
        out_shape=jax.ShapeDtypeStruct((NPAD, D), jnp.float32),
    )


@functools.lru_cache(maxsize=1)
def _make_segsum():
    mesh = plsc.VectorSubcoreMesh(core_axis_name="c", subcore_axis_name="s",
                                  num_cores=2, num_subcores=16)

    @functools.partial(
        pl.kernel,
        mesh=mesh,
        out_type=jax.ShapeDtypeStruct((2, NPAD, D), jnp.float32),
        scratch_types=[
            pltpu.VMEM((CPP, K), jnp.int32),
            pltpu.VMEM((CPP, K), jnp.int32),
            pltpu.VMEM((K, D), jnp.float32),
            pltpu.VMEM((K, D), jnp.float32),
            pltpu.VMEM_SHARED((NPAD, D), jnp.float32),
            pltpu.SemaphoreType.DMA,
            pltpu.SemaphoreType.DMA,
        ],
    )
    def segsum(h_hbm, src_hbm, dst_hbm, out_hbm,
               src_v, dst_v, rows0, rows1, acc, sem0, sem1):
        cid = lax.axis_index("c")
        sid = lax.axis_index("s")
        wid = cid * 16 + sid
        rpt = NPAD // 16
        rbase = sid * rpt
        pltpu.sync_copy(h_hbm.at[pl.ds(rbase, rpt)], acc.at[pl.ds(rbase, rpt)])
        plsc.subcore_barrier()

        for phase in range(PHASES):
            pbase = phase * CPP
            pltpu.sync_copy(src_hbm.at[wid, pl.ds(pbase, CPP)], src_v)
            pltpu.sync_copy(dst_hbm.at[wid, pl.ds(pbase, CPP)], dst_v)
            pltpu.async_copy(h_hbm.at[src_v.at[0]], rows0, sem0)

            def body(i, carry):
                j = i * 2
                pltpu.async_copy(h_hbm.at[src_v.at[j + 1]], rows1, sem1)
                pltpu.make_async_copy(h_hbm.at[src_v.at[j]], rows0, sem0).wait()
                pltpu.sync_copy(rows0, acc.at[dst_v.at[j]], add=True)

                @pl.when(j + 2 < CPP)
                def _():
                    pltpu.async_copy(h_hbm.at[src_v.at[j + 2]], rows0, sem0)

                pltpu.make_async_copy(h_hbm.at[src_v.at[j + 1]], rows1,
                                      sem1).wait()
                pltpu.sync_copy(rows1, acc.at[dst_v.at[j + 1]], add=True)
                return carry

            lax.fori_loop(0, CPP // 2, body, 0)
        plsc.subcore_barrier()
        pltpu.sync_copy(acc.at[pl.ds(rbase, rpt)],
                        out_hbm.at[cid, pl.ds(rbase, rpt)])

    return segsum


def kernel(x, edge_index, params):
    xp = jnp.pad(x, ((0, NPAD - N), (0, 0)))

    def row(v):
        return v.reshape(1, -1)

    p = params
    h = _tc_call(_in_proj_body, 1,
                 [(D, 512), (1, 512), (512, D), (1, D), (1, D), (1, D)])(
        xp, p['in_w1'].T, row(p['in_b1']), p['in_w2'].T, row(p['in_b2']),
        row(p['in_ln_g']), row(p['in_ln_b']))

    pad_len = EPAD - E
    pad_idx = N + (jnp.arange(pad_len, dtype=jnp.int32) % (NPAD - N))
    srcp = jnp.concatenate([edge_index[0], pad_idx]).reshape(NW, CHUNKS, K)
    dstp = jnp.concatenate([edge_index[1], pad_idx]).reshape(NW, CHUNKS, K)

    layer_call = _tc_call(
        _layer_body, 1,
        [(D, D), (1, D), (D, D), (1, D), (1, D), (1, D)],
        extra_specs=[pl.BlockSpec((2, RB, D), lambda i: (0, i, 0))])

    segsum = _make_segsum()
    for lp in p['layers']:
        ab = segsum(h, srcp, dstp)
        h = layer_call(h, ab, lp['w1'].T, row(lp['b1']), lp['w2'].T,
                       row(lp['b2']), row(lp['ln_g']), row(lp['ln_b']))

    out = _tc_call(_out_proj_body, 2,
                   [(D, 512), (1, 512), (512, D), (1, D), (1, D), (1, D)])(
        h, xp, p['d_w1'].T, row(p['d_b1']), p['d_w2'].T, row(p['d_b2']),
        row(p['out_ln_g']), row(p['out_ln_b']))
    return out[:N]
